# baseline (device time: 178951 ns/iter reference)
import jax
import jax.numpy as jnp
from jax import lax
from jax.experimental import pallas as pl
from jax.experimental.pallas import tpu as pltpu

N_DEV = 4
SQ = 512
SKV = 2048
D = 1024
HQ = 8
DH = 128
SCALE = 0.08838834764831843


def _attn_partial(xc, wq_ref, wo_ref, k_ref, v_ref):
    q = jnp.dot(xc, wq_ref[...],
                preferred_element_type=jnp.float32).astype(jnp.bfloat16)
    cols = []
    for hh in range(HQ):
        qh = q[:, hh * DH:(hh + 1) * DH]
        kh = k_ref[:, hh * DH:(hh + 1) * DH]
        vh = v_ref[:, hh * DH:(hh + 1) * DH]
        s = lax.dot_general(qh, kh, (((1,), (1,)), ((), ())),
                            preferred_element_type=jnp.float32) * SCALE
        e = jnp.exp(s)
        l = jnp.sum(e, axis=1, keepdims=True)
        oh = lax.dot_general(e.astype(jnp.bfloat16), vh,
                             (((1,), (0,)), ((), ())),
                             preferred_element_type=jnp.float32)
        cols.append((oh / l).astype(jnp.bfloat16))
    attn = jnp.concatenate(cols, axis=1)
    return jnp.dot(attn, wo_ref[...], preferred_element_type=jnp.float32)


def _body(x_ref, wq_ref, wo_ref, k_ref, v_ref, out_ref,
          ag_buf, rs_send_buf, rs_recv_buf,
          ag_send, ag_recv, rs_send, rs_recv):
    j = lax.axis_index("i")
    left = (j + N_DEV - 1) % N_DEV
    right = (j + 1) % N_DEV

    barrier = pltpu.get_barrier_semaphore()
    for nbr in (left, right):
        pl.semaphore_signal(barrier, inc=1, device_id=(nbr,),
                            device_id_type=pl.DeviceIdType.MESH)
    pl.semaphore_wait(barrier, 2)

    ag_buf[0, :, :] = x_ref[...]
    ag = [
        pltpu.make_async_remote_copy(
            src_ref=ag_buf.at[h], dst_ref=ag_buf.at[h + 1],
            send_sem=ag_send.at[h], recv_sem=ag_recv.at[h],
            device_id=(right,), device_id_type=pl.DeviceIdType.MESH)
        for h in range(N_DEV - 1)
    ]
    rs = [
        pltpu.make_async_remote_copy(
            src_ref=rs_send_buf.at[s], dst_ref=rs_recv_buf.at[s],
            send_sem=rs_send.at[s], recv_sem=rs_recv.at[s],
            device_id=(right,), device_id_type=pl.DeviceIdType.MESH)
        for s in range(N_DEV - 1)
    ]

    ag[0].start()
    out_ref[...] = _attn_partial(x_ref[...], wq_ref, wo_ref, k_ref, v_ref)

    ag[0].wait_recv()
    ag[1].start()
    p = _attn_partial(ag_buf[1], wq_ref, wo_ref, k_ref, v_ref)
    rs_send_buf[0, :, :] = p.astype(jnp.bfloat16)
    rs[0].start()

    ag[1].wait_recv()
    ag[2].start()
    p = _attn_partial(ag_buf[2], wq_ref, wo_ref, k_ref, v_ref)
    rs[0].wait_recv()
    rs_send_buf[1, :, :] = (
        p + rs_recv_buf[0].astype(jnp.float32)).astype(jnp.bfloat16)
    rs[1].start()

    ag[2].wait_recv()
    p = _attn_partial(ag_buf[3], wq_ref, wo_ref, k_ref, v_ref)
    rs[1].wait_recv()
    rs_send_buf[2, :, :] = (
        p + rs_recv_buf[1].astype(jnp.float32)).astype(jnp.bfloat16)
    rs[2].start()

    rs[2].wait_recv()
    out_ref[...] = out_ref[...] + rs_recv_buf[2].astype(jnp.float32)

    for r in ag + rs:
        r.wait_send()


def kernel(x, Wq, Wo, K_ext, V_ext):
    j = lax.axis_index("i")
    xb = x[0].astype(jnp.bfloat16)
    wq = Wq.astype(jnp.bfloat16)
    wo = Wo.astype(jnp.bfloat16)
    k2 = K_ext.reshape(SKV, 32 * DH)
    v2 = V_ext.reshape(SKV, 32 * DH)
    kb = lax.dynamic_slice_in_dim(k2, j * HQ * DH, HQ * DH,
                                  axis=1).astype(jnp.bfloat16)
    vb = lax.dynamic_slice_in_dim(v2, j * HQ * DH, HQ * DH,
                                  axis=1).astype(jnp.bfloat16)

    out = pl.pallas_call(
        _body,
        out_shape=jax.ShapeDtypeStruct((SQ, D), jnp.float32),
        in_specs=[pl.BlockSpec(memory_space=pltpu.VMEM)] * 5,
        out_specs=pl.BlockSpec(memory_space=pltpu.VMEM),
        scratch_shapes=[
            pltpu.VMEM((N_DEV, SQ, D), jnp.bfloat16),
            pltpu.VMEM((N_DEV - 1, SQ, D), jnp.bfloat16),
            pltpu.VMEM((N_DEV - 1, SQ, D), jnp.bfloat16),
            pltpu.SemaphoreType.DMA((N_DEV - 1,)),
            pltpu.SemaphoreType.DMA((N_DEV - 1,)),
            pltpu.SemaphoreType.DMA((N_DEV - 1,)),
            pltpu.SemaphoreType.DMA((N_DEV - 1,)),
        ],
        compiler_params=pltpu.CompilerParams(
            collective_id=0, vmem_limit_bytes=100 * 1024 * 1024),
    )(xb, wq, wo, kb, vb)
    return out.reshape(1, SQ, D)


# device time: 113218 ns/iter; 1.5806x vs baseline; 1.5806x over previous
import jax
import jax.numpy as jnp
from jax import lax
from jax.experimental import pallas as pl
from jax.experimental.pallas import tpu as pltpu

N_DEV = 4
SQ = 512
SKV = 2048
D = 1024
HQ = 8
DH = 128
SCALE = 0.08838834764831843


def _attn_partial(xc, wq_ref, wo_ref, k_ref, v_ref):
    q = jnp.dot(xc, wq_ref[...],
                preferred_element_type=jnp.float32).astype(jnp.bfloat16)
    cols = []
    for hh in range(HQ):
        qh = q[:, hh * DH:(hh + 1) * DH]
        s = lax.dot_general(qh, k_ref[hh], (((1,), (1,)), ((), ())),
                            preferred_element_type=jnp.float32) * SCALE
        e = jnp.exp(s)
        l = jnp.sum(e, axis=1, keepdims=True)
        oh = lax.dot_general(e.astype(jnp.bfloat16), v_ref[hh],
                             (((1,), (0,)), ((), ())),
                             preferred_element_type=jnp.float32)
        cols.append((oh / l).astype(jnp.bfloat16))
    attn = jnp.concatenate(cols, axis=1)
    return jnp.dot(attn, wo_ref[...], preferred_element_type=jnp.float32)


def _body(x_ref, wq_ref, wo_ref, k_ref, v_ref, out_ref,
          ag_buf, rs_send_buf, rs_recv_buf,
          ag_send, ag_recv, rs_send, rs_recv):
    j = lax.axis_index("i")
    left = (j + N_DEV - 1) % N_DEV
    right = (j + 1) % N_DEV

    barrier = pltpu.get_barrier_semaphore()
    for nbr in (left, right):
        pl.semaphore_signal(barrier, inc=1, device_id=(nbr,),
                            device_id_type=pl.DeviceIdType.MESH)
    pl.semaphore_wait(barrier, 2)

    ag_buf[0, :, :] = x_ref[...]
    ag = [
        pltpu.make_async_remote_copy(
            src_ref=ag_buf.at[h], dst_ref=ag_buf.at[h + 1],
            send_sem=ag_send.at[h], recv_sem=ag_recv.at[h],
            device_id=(right,), device_id_type=pl.DeviceIdType.MESH)
        for h in range(N_DEV - 1)
    ]
    rs = [
        pltpu.make_async_remote_copy(
            src_ref=rs_send_buf.at[s], dst_ref=rs_recv_buf.at[s],
            send_sem=rs_send.at[s], recv_sem=rs_recv.at[s],
            device_id=(right,), device_id_type=pl.DeviceIdType.MESH)
        for s in range(N_DEV - 1)
    ]

    ag[0].start()
    out_ref[...] = _attn_partial(x_ref[...], wq_ref, wo_ref, k_ref, v_ref)

    ag[0].wait_recv()
    ag[1].start()
    p = _attn_partial(ag_buf[1], wq_ref, wo_ref, k_ref, v_ref)
    rs_send_buf[0, :, :] = p.astype(jnp.bfloat16)
    rs[0].start()

    ag[1].wait_recv()
    ag[2].start()
    p = _attn_partial(ag_buf[2], wq_ref, wo_ref, k_ref, v_ref)
    rs[0].wait_recv()
    rs_send_buf[1, :, :] = (
        p + rs_recv_buf[0].astype(jnp.float32)).astype(jnp.bfloat16)
    rs[1].start()

    ag[2].wait_recv()
    p = _attn_partial(ag_buf[3], wq_ref, wo_ref, k_ref, v_ref)
    rs[1].wait_recv()
    rs_send_buf[2, :, :] = (
        p + rs_recv_buf[1].astype(jnp.float32)).astype(jnp.bfloat16)
    rs[2].start()

    rs[2].wait_recv()
    out_ref[...] = out_ref[...] + rs_recv_buf[2].astype(jnp.float32)

    for r in ag + rs:
        r.wait_send()


def kernel(x, Wq, Wo, K_ext, V_ext):
    j = lax.axis_index("i")
    xb = x[0].astype(jnp.bfloat16)
    wq = Wq.astype(jnp.bfloat16)
    wo = Wo.astype(jnp.bfloat16)
    k = lax.dynamic_slice_in_dim(K_ext[0], j * HQ, HQ, axis=1)
    v = lax.dynamic_slice_in_dim(V_ext[0], j * HQ, HQ, axis=1)
    kb = jnp.transpose(k, (1, 0, 2)).astype(jnp.bfloat16)
    vb = jnp.transpose(v, (1, 0, 2)).astype(jnp.bfloat16)

    out = pl.pallas_call(
        _body,
        out_shape=jax.ShapeDtypeStruct((SQ, D), jnp.float32),
        in_specs=[pl.BlockSpec(memory_space=pltpu.VMEM)] * 5,
        out_specs=pl.BlockSpec(memory_space=pltpu.VMEM),
        scratch_shapes=[
            pltpu.VMEM((N_DEV, SQ, D), jnp.bfloat16),
            pltpu.VMEM((N_DEV - 1, SQ, D), jnp.bfloat16),
            pltpu.VMEM((N_DEV - 1, SQ, D), jnp.bfloat16),
            pltpu.SemaphoreType.DMA((N_DEV - 1,)),
            pltpu.SemaphoreType.DMA((N_DEV - 1,)),
            pltpu.SemaphoreType.DMA((N_DEV - 1,)),
            pltpu.SemaphoreType.DMA((N_DEV - 1,)),
        ],
        compiler_params=pltpu.CompilerParams(
            collective_id=0, vmem_limit_bytes=100 * 1024 * 1024),
    )(xb, wq, wo, kb, vb)
    return out.reshape(1, SQ, D)


# device time: 112227 ns/iter; 1.5945x vs baseline; 1.0088x over previous
import jax
import jax.numpy as jnp
from jax import lax
from jax.experimental import pallas as pl
from jax.experimental.pallas import tpu as pltpu

N_DEV = 4
SQ = 512
SKV = 2048
D = 1024
HQ = 8
DH = 128
SCALE_LOG2E = 0.08838834764831843 * 1.4426950408889634


def _attn_partial(xc, wq_ref, wo_ref, k_ref, v_ref):
    q = (jnp.dot(xc, wq_ref[...], preferred_element_type=jnp.float32)
         * SCALE_LOG2E).astype(jnp.bfloat16)
    cols = []
    for hh in range(HQ):
        qh = q[:, hh * DH:(hh + 1) * DH]
        s = lax.dot_general(qh, k_ref[hh], (((1,), (1,)), ((), ())),
                            preferred_element_type=jnp.float32)
        e = jnp.exp2(s)
        l = jnp.sum(e, axis=1, keepdims=True)
        oh = lax.dot_general(e.astype(jnp.bfloat16), v_ref[hh],
                             (((1,), (0,)), ((), ())),
                             preferred_element_type=jnp.float32)
        cols.append((oh / l).astype(jnp.bfloat16))
    attn = jnp.concatenate(cols, axis=1)
    return jnp.dot(attn, wo_ref[...], preferred_element_type=jnp.float32)


def _body(x_ref, wq_ref, wo_ref, k_ref, v_ref, out_ref,
          ag_buf, rs_send_buf, rs_recv_buf,
          ag_send, ag_recv, rs_send, rs_recv):
    j = lax.axis_index("i")
    left = (j + N_DEV - 1) % N_DEV
    right = (j + 1) % N_DEV

    barrier = pltpu.get_barrier_semaphore()
    for nbr in (left, right):
        pl.semaphore_signal(barrier, inc=1, device_id=(nbr,),
                            device_id_type=pl.DeviceIdType.MESH)
    pl.semaphore_wait(barrier, 2)

    ag_buf[0, :, :] = x_ref[...]
    ag = [
        pltpu.make_async_remote_copy(
            src_ref=ag_buf.at[h], dst_ref=ag_buf.at[h + 1],
            send_sem=ag_send.at[h], recv_sem=ag_recv.at[h],
            device_id=(right,), device_id_type=pl.DeviceIdType.MESH)
        for h in range(N_DEV - 1)
    ]
    rs = [
        pltpu.make_async_remote_copy(
            src_ref=rs_send_buf.at[s], dst_ref=rs_recv_buf.at[s],
            send_sem=rs_send.at[s], recv_sem=rs_recv.at[s],
            device_id=(right,), device_id_type=pl.DeviceIdType.MESH)
        for s in range(N_DEV - 1)
    ]

    ag[0].start()
    out_ref[...] = _attn_partial(x_ref[...], wq_ref, wo_ref, k_ref, v_ref)

    ag[0].wait_recv()
    ag[1].start()
    p = _attn_partial(ag_buf[1], wq_ref, wo_ref, k_ref, v_ref)
    rs_send_buf[0, :, :] = p.astype(jnp.bfloat16)
    rs[0].start()

    ag[1].wait_recv()
    ag[2].start()
    p = _attn_partial(ag_buf[2], wq_ref, wo_ref, k_ref, v_ref)
    rs[0].wait_recv()
    rs_send_buf[1, :, :] = (
        p + rs_recv_buf[0].astype(jnp.float32)).astype(jnp.bfloat16)
    rs[1].start()

    ag[2].wait_recv()
    p = _attn_partial(ag_buf[3], wq_ref, wo_ref, k_ref, v_ref)
    rs[1].wait_recv()
    rs_send_buf[2, :, :] = (
        p + rs_recv_buf[1].astype(jnp.float32)).astype(jnp.bfloat16)
    rs[2].start()

    rs[2].wait_recv()
    out_ref[...] = out_ref[...] + rs_recv_buf[2].astype(jnp.float32)

    for r in ag + rs:
        r.wait_send()


def kernel(x, Wq, Wo, K_ext, V_ext):
    j = lax.axis_index("i")
    xb = x[0].astype(jnp.bfloat16)
    wq = Wq.astype(jnp.bfloat16)
    wo = Wo.astype(jnp.bfloat16)
    k = lax.dynamic_slice_in_dim(K_ext[0], j * HQ, HQ, axis=1)
    v = lax.dynamic_slice_in_dim(V_ext[0], j * HQ, HQ, axis=1)
    kb = jnp.transpose(k, (1, 0, 2)).astype(jnp.bfloat16)
    vb = jnp.transpose(v, (1, 0, 2)).astype(jnp.bfloat16)

    out = pl.pallas_call(
        _body,
        out_shape=jax.ShapeDtypeStruct((SQ, D), jnp.float32),
        in_specs=[pl.BlockSpec(memory_space=pltpu.VMEM)] * 5,
        out_specs=pl.BlockSpec(memory_space=pltpu.VMEM),
        scratch_shapes=[
            pltpu.VMEM((N_DEV, SQ, D), jnp.bfloat16),
            pltpu.VMEM((N_DEV - 1, SQ, D), jnp.bfloat16),
            pltpu.VMEM((N_DEV - 1, SQ, D), jnp.bfloat16),
            pltpu.SemaphoreType.DMA((N_DEV - 1,)),
            pltpu.SemaphoreType.DMA((N_DEV - 1,)),
            pltpu.SemaphoreType.DMA((N_DEV - 1,)),
            pltpu.SemaphoreType.DMA((N_DEV - 1,)),
        ],
        compiler_params=pltpu.CompilerParams(
            collective_id=0, vmem_limit_bytes=100 * 1024 * 1024),
    )(xb, wq, wo, kb, vb)
    return out.reshape(1, SQ, D)


# device time: 111671 ns/iter; 1.6025x vs baseline; 1.0050x over previous
import jax
import jax.numpy as jnp
from jax import lax
from jax.experimental import pallas as pl
from jax.experimental.pallas import tpu as pltpu

N_DEV = 4
SQ = 512
SKV = 2048
D = 1024
HQ = 8
DH = 128
SCALE_LOG2E = 0.08838834764831843 * 1.4426950408889634


def _attn_partial(xc, wq_ref, wo_ref, k_ref, v_ref):
    q = (jnp.dot(xc, wq_ref[...], preferred_element_type=jnp.float32)
         * SCALE_LOG2E).astype(jnp.bfloat16)
    cols = []
    for hh in range(HQ):
        qh = q[:, hh * DH:(hh + 1) * DH]
        s = lax.dot_general(qh, k_ref[hh], (((1,), (1,)), ((), ())),
                            preferred_element_type=jnp.float32)
        e = jnp.exp2(s)
        l = jnp.sum(e, axis=1, keepdims=True)
        oh = lax.dot_general(e.astype(jnp.bfloat16), v_ref[hh],
                             (((1,), (0,)), ((), ())),
                             preferred_element_type=jnp.float32)
        cols.append((oh / l).astype(jnp.bfloat16))
    attn = jnp.concatenate(cols, axis=1)
    return jnp.dot(attn, wo_ref[...], preferred_element_type=jnp.float32)


def _body(x_ref, wq_ref, wo_ref, k_ref, v_ref, out_ref,
          xg, psend, pr,
          x_send, x_recv, p_send, p_recv):
    j = lax.axis_index("i")

    barrier = pltpu.get_barrier_semaphore()
    for k in range(1, N_DEV):
        pl.semaphore_signal(barrier, inc=1, device_id=((j + k) % N_DEV,),
                            device_id_type=pl.DeviceIdType.MESH)
    pl.semaphore_wait(barrier, N_DEV - 1)

    dx = [
        pltpu.make_async_remote_copy(
            src_ref=x_ref, dst_ref=xg.at[k - 1],
            send_sem=x_send.at[k - 1], recv_sem=x_recv.at[k - 1],
            device_id=((j + k) % N_DEV,),
            device_id_type=pl.DeviceIdType.MESH)
        for k in range(1, N_DEV)
    ]
    dp = [
        pltpu.make_async_remote_copy(
            src_ref=psend.at[k - 1], dst_ref=pr.at[k - 1],
            send_sem=p_send.at[k - 1], recv_sem=p_recv.at[k - 1],
            device_id=((j + k) % N_DEV,),
            device_id_type=pl.DeviceIdType.MESH)
        for k in range(1, N_DEV)
    ]

    for d in dx:
        d.start()
    out_ref[...] = _attn_partial(x_ref[...], wq_ref, wo_ref, k_ref, v_ref)

    dx[2].wait_recv()
    psend[0, :, :] = _attn_partial(
        xg[2], wq_ref, wo_ref, k_ref, v_ref).astype(jnp.bfloat16)
    dp[0].start()

    dx[0].wait_recv()
    psend[2, :, :] = _attn_partial(
        xg[0], wq_ref, wo_ref, k_ref, v_ref).astype(jnp.bfloat16)
    dp[2].start()

    dx[1].wait_recv()
    psend[1, :, :] = _attn_partial(
        xg[1], wq_ref, wo_ref, k_ref, v_ref).astype(jnp.bfloat16)
    dp[1].start()

    for d in dp:
        d.wait_recv()
    out_ref[...] = (out_ref[...]
                    + pr[0].astype(jnp.float32)
                    + pr[1].astype(jnp.float32)
                    + pr[2].astype(jnp.float32))

    for d in dx + dp:
        d.wait_send()


def kernel(x, Wq, Wo, K_ext, V_ext):
    j = lax.axis_index("i")
    xb = x[0].astype(jnp.bfloat16)
    wq = Wq.astype(jnp.bfloat16)
    wo = Wo.astype(jnp.bfloat16)
    k = lax.dynamic_slice_in_dim(K_ext[0], j * HQ, HQ, axis=1)
    v = lax.dynamic_slice_in_dim(V_ext[0], j * HQ, HQ, axis=1)
    kb = jnp.transpose(k, (1, 0, 2)).astype(jnp.bfloat16)
    vb = jnp.transpose(v, (1, 0, 2)).astype(jnp.bfloat16)

    out = pl.pallas_call(
        _body,
        out_shape=jax.ShapeDtypeStruct((SQ, D), jnp.float32),
        in_specs=[pl.BlockSpec(memory_space=pltpu.VMEM)] * 5,
        out_specs=pl.BlockSpec(memory_space=pltpu.VMEM),
        scratch_shapes=[
            pltpu.VMEM((N_DEV - 1, SQ, D), jnp.bfloat16),
            pltpu.VMEM((N_DEV - 1, SQ, D), jnp.bfloat16),
            pltpu.VMEM((N_DEV - 1, SQ, D), jnp.bfloat16),
            pltpu.SemaphoreType.DMA((N_DEV - 1,)),
            pltpu.SemaphoreType.DMA((N_DEV - 1,)),
            pltpu.SemaphoreType.DMA((N_DEV - 1,)),
            pltpu.SemaphoreType.DMA((N_DEV - 1,)),
        ],
        compiler_params=pltpu.CompilerParams(
            collective_id=0, vmem_limit_bytes=100 * 1024 * 1024),
    )(xb, wq, wo, kb, vb)
    return out.reshape(1, SQ, D)
